# baseline (speedup 1.0000x reference)
import functools

import jax
import jax.numpy as jnp
from jax import lax
from jax.experimental import pallas as pl
from jax.experimental.pallas import tpu as pltpu
from jax.experimental.pallas import tpu_sc as plsc

B = 4096
D = 1024
EPS = 1e-12


BM = 512
NJ = B // BM
NCK = 4
CK = B // NCK


def _fused_body(x_ref, idx_ref, xn_sc, sqc_sc):
    p = pl.program_id(0)
    j = pl.program_id(1)

    @pl.when(p == 0)
    def _norm():
        xc = x_ref[...]
        n2 = jnp.sum(xc * xc, axis=1, keepdims=True)
        xn = xc / jnp.maximum(jnp.sqrt(n2), EPS)
        xn_sc[pl.ds(j * BM, BM), :] = xn
        sq = jnp.sum(xn * xn, axis=1, keepdims=True)
        sqc_sc[:, pl.ds(j * BM, BM)] = sq.T

    @pl.when(p == 1)
    def _dist():
        a = xn_sc[pl.ds(j * BM, BM), :]
        sqr = jnp.sum(a * a, axis=1, keepdims=True)
        a2 = a * (-2.0)
        big = jnp.int32(2 * B)
        s_run = None
        loc_run = None
        for c in range(NCK):
            g2 = lax.dot_general(
                a2, xn_sc[pl.ds(c * CK, CK), :],
                dimension_numbers=(((1,), (1,)), ((), ())),
                preferred_element_type=jnp.float32,
            )
            d2_c = (sqr + sqc_sc[:, pl.ds(c * CK, CK)]) + g2
            m2_c = jnp.max(d2_c, axis=1, keepdims=True)
            mt = m2_c.T
            s_c = jnp.sqrt(jnp.maximum(mt, 0.0))
            t = mt
            for _ in range(6):
                cand = lax.bitcast_convert_type(
                    lax.bitcast_convert_type(t, jnp.int32) - 1,
                    jnp.float32)
                t = jnp.where(jnp.sqrt(cand) == s_c, cand, t)
            col = lax.broadcasted_iota(jnp.int32, (BM, CK), 1) + c * CK
            loc_c = jnp.min(
                jnp.where(d2_c >= t.T, col, big), axis=1, keepdims=True)
            sc_col = s_c.T
            if s_run is None:
                s_run, loc_run = sc_col, loc_c
            else:
                better = sc_col > s_run
                loc_run = jnp.where(better, loc_c, loc_run)
                s_run = jnp.maximum(s_run, sc_col)
        idx_ref[...] = loc_run


def _argmax_fused(x):
    return pl.pallas_call(
        _fused_body,
        grid=(2, NJ),
        in_specs=[pl.BlockSpec((BM, D), lambda p, j: (j, 0))],
        out_specs=pl.BlockSpec((BM, 1), lambda p, j: (j, 0)),
        out_shape=jax.ShapeDtypeStruct((B, 1), jnp.int32),
        scratch_shapes=[
            pltpu.VMEM((B, D), jnp.float32),
            pltpu.VMEM((1, B), jnp.float32),
        ],
        compiler_params=pltpu.CompilerParams(
            dimension_semantics=("arbitrary", "arbitrary"),
        ),
    )(x)


_NC = 2
_NS = 16
_NW = _NC * _NS
_BPW = B // _NW
_CH = 16
_NV = D // 16


def _sc_axpy_body(x_hbm, idx_hbm, out_hbm, part_hbm,
                  idx_v, acc_v, x_v, xg_v, sem):
    kinv = jnp.float32(B / (B + 1.0))
    wid = lax.axis_index("s") * _NC + lax.axis_index("c")
    base = wid * _BPW
    pltpu.sync_copy(idx_hbm.at[pl.ds(base, _BPW)], idx_v)

    def zacc(v, _):
        acc_v[pl.ds(v * 16, 16)] = jnp.zeros((16,), jnp.float32)
        return _

    lax.fori_loop(0, _NV, zacc, 0)

    def chunk(c, _):
        off = c * _CH
        cp = pltpu.async_copy(
            x_hbm.at[idx_v.at[pl.ds(off, _CH)]], xg_v, sem)
        pltpu.sync_copy(x_hbm.at[pl.ds(base + off, _CH)], x_v)
        cp.wait()

        def dloop(dv, _):
            o = pl.ds(dv * 16, 16)

            def rloop(r, acc):
                xg = xg_v[r, o]
                x_v[r, o] = x_v[r, o] - kinv * xg
                return acc + xg

            acc = lax.fori_loop(0, _CH, rloop, acc_v[o])
            acc_v[o] = acc
            return _

        lax.fori_loop(0, _NV, dloop, 0)
        pltpu.sync_copy(x_v, out_hbm.at[pl.ds(base + off, _CH)])
        return _

    lax.fori_loop(0, _BPW // _CH, chunk, 0)
    pltpu.sync_copy(acc_v, part_hbm.at[wid])


def _sc_axpy(x, idx):
    mesh = plsc.VectorSubcoreMesh(core_axis_name="c", subcore_axis_name="s")
    k = functools.partial(
        pl.kernel,
        mesh=mesh,
        out_type=(
            jax.ShapeDtypeStruct((B, D), jnp.float32),
            jax.ShapeDtypeStruct((_NW, D), jnp.float32),
        ),
        scratch_types=[
            pltpu.VMEM((_BPW,), jnp.int32),
            pltpu.VMEM((D,), jnp.float32),
            pltpu.VMEM((_CH, D), jnp.float32),
            pltpu.VMEM((_CH, D), jnp.float32),
            pltpu.SemaphoreType.DMA,
        ],
    )(_sc_axpy_body)
    return k(x, idx)


BM_EP = 512


def _final_body(o0_ref, part_ref, o_ref, s_sc):
    inv = jnp.float32(1.0 / (B + 1.0))

    @pl.when(pl.program_id(0) == 0)
    def _init():
        s_sc[...] = jnp.sum(part_ref[...], axis=0, keepdims=True) * inv

    o_ref[...] = o0_ref[...] + s_sc[...]


def _final(o0, part):
    return pl.pallas_call(
        _final_body,
        grid=(B // BM_EP,),
        in_specs=[
            pl.BlockSpec((BM_EP, D), lambda i: (i, 0)),
            pl.BlockSpec((_NW, D), lambda i: (0, 0)),
        ],
        out_specs=pl.BlockSpec((BM_EP, D), lambda i: (i, 0)),
        out_shape=jax.ShapeDtypeStruct((B, D), jnp.float32),
        scratch_shapes=[pltpu.VMEM((1, D), jnp.float32)],
        compiler_params=pltpu.CompilerParams(
            dimension_semantics=("arbitrary",),
        ),
    )(o0, part)


def kernel(x):
    idx2 = _argmax_fused(x)
    idx = idx2.reshape(B)
    o0, part = _sc_axpy(x, idx)
    return _final(o0, part)


# speedup vs baseline: 1.3792x; 1.3792x over previous
import functools

import jax
import jax.numpy as jnp
from jax import lax
from jax.experimental import pallas as pl
from jax.experimental.pallas import tpu as pltpu
from jax.experimental.pallas import tpu_sc as plsc

B = 4096
D = 1024
EPS = 1e-12


BM = 512
NJ = B // BM
NCK = 4
CK = B // NCK


def _fused_body(x_ref, idx_ref, xn_sc, sqc_sc):
    p = pl.program_id(0)
    j = pl.program_id(1)

    @pl.when(p == 0)
    def _norm():
        xc = x_ref[...]
        n2 = jnp.sum(xc * xc, axis=1, keepdims=True)
        xn = xc / jnp.maximum(jnp.sqrt(n2), EPS)
        xn_sc[pl.ds(j * BM, BM), :] = xn
        sq = jnp.sum(xn * xn, axis=1, keepdims=True)
        sqc_sc[:, pl.ds(j * BM, BM)] = sq.T

    @pl.when(p == 1)
    def _dist():
        a = xn_sc[pl.ds(j * BM, BM), :]
        sqr = jnp.sum(a * a, axis=1, keepdims=True)
        a2 = a * (-2.0)
        big = jnp.int32(2 * B)
        s_run = None
        loc_run = None
        for c in range(NCK):
            g2 = lax.dot_general(
                a2, xn_sc[pl.ds(c * CK, CK), :],
                dimension_numbers=(((1,), (1,)), ((), ())),
                preferred_element_type=jnp.float32,
            )
            d2_c = (sqr + sqc_sc[:, pl.ds(c * CK, CK)]) + g2
            m2_c = jnp.max(d2_c, axis=1, keepdims=True)
            mt = m2_c.T
            s_c = jnp.sqrt(jnp.maximum(mt, 0.0))
            t = mt
            for _ in range(6):
                cand = lax.bitcast_convert_type(
                    lax.bitcast_convert_type(t, jnp.int32) - 1,
                    jnp.float32)
                t = jnp.where(jnp.sqrt(cand) == s_c, cand, t)
            col = lax.broadcasted_iota(jnp.int32, (BM, CK), 1) + c * CK
            loc_c = jnp.min(
                jnp.where(d2_c >= t.T, col, big), axis=1, keepdims=True)
            sc_col = s_c.T
            if s_run is None:
                s_run, loc_run = sc_col, loc_c
            else:
                better = sc_col > s_run
                loc_run = jnp.where(better, loc_c, loc_run)
                s_run = jnp.maximum(s_run, sc_col)
        idx_ref[...] = loc_run


def _argmax_fused(x):
    return pl.pallas_call(
        _fused_body,
        grid=(2, NJ),
        in_specs=[pl.BlockSpec((BM, D), lambda p, j: (j, 0))],
        out_specs=pl.BlockSpec((BM, 1), lambda p, j: (j, 0)),
        out_shape=jax.ShapeDtypeStruct((B, 1), jnp.int32),
        scratch_shapes=[
            pltpu.VMEM((B, D), jnp.float32),
            pltpu.VMEM((1, B), jnp.float32),
        ],
        compiler_params=pltpu.CompilerParams(
            dimension_semantics=("arbitrary", "arbitrary"),
        ),
    )(x)


_NC = 2
_NS = 16
_NW = _NC * _NS
_BPW = B // _NW
_CH = 64


def _sc_gather_body(table_hbm, idx_hbm, out_hbm, idx_v, rows_v, sem):
    wid = lax.axis_index("s") * _NC + lax.axis_index("c")
    base = wid * _BPW
    pltpu.sync_copy(idx_hbm.at[pl.ds(base, _BPW)], idx_v)

    def chunk(c, _):
        off = c * _CH
        pltpu.async_copy(
            table_hbm.at[idx_v.at[pl.ds(off, _CH)]],
            rows_v, sem).wait()
        pltpu.sync_copy(rows_v, out_hbm.at[pl.ds(base + off, _CH)])
        return _

    lax.fori_loop(0, _BPW // _CH, chunk, 0)


def _sc_gather(x, idx):
    mesh = plsc.VectorSubcoreMesh(core_axis_name="c", subcore_axis_name="s")
    k = functools.partial(
        pl.kernel,
        mesh=mesh,
        out_type=jax.ShapeDtypeStruct((B, D), jnp.float32),
        scratch_types=[
            pltpu.VMEM((_BPW,), jnp.int32),
            pltpu.VMEM((_CH, D), jnp.float32),
            pltpu.SemaphoreType.DMA,
        ],
    )(_sc_gather_body)
    return k(x, idx)


BM_EP = 512


def _ep_body(xg_ref, x_ref, o_ref, s_sc):
    p = pl.program_id(0)
    kf = jnp.float32(B)
    inv = jnp.float32(1.0 / (B + 1.0))

    @pl.when((p == 0) & (pl.program_id(1) == 0))
    def _init():
        s_sc[...] = jnp.zeros((1, D), jnp.float32)

    @pl.when(p == 0)
    def _sum():
        s_sc[...] += jnp.sum(xg_ref[...], axis=0, keepdims=True)

    @pl.when(p == 1)
    def _out():
        o_ref[...] = x_ref[...] + (s_sc[...] - kf * xg_ref[...]) * inv


def _epilogue(xg, x):
    return pl.pallas_call(
        _ep_body,
        grid=(2, B // BM_EP),
        in_specs=[
            pl.BlockSpec((BM_EP, D), lambda p, i: (i, 0)),
            pl.BlockSpec((BM_EP, D), lambda p, i: (p * i, 0)),
        ],
        out_specs=pl.BlockSpec((BM_EP, D), lambda p, i: (p * i, 0)),
        out_shape=jax.ShapeDtypeStruct((B, D), jnp.float32),
        scratch_shapes=[pltpu.VMEM((1, D), jnp.float32)],
        compiler_params=pltpu.CompilerParams(
            dimension_semantics=("arbitrary", "arbitrary"),
        ),
    )(xg, x)


def kernel(x):
    idx2 = _argmax_fused(x)
    idx = idx2.reshape(B)
    xg = _sc_gather(x, idx)
    return _epilogue(xg, x)
